# Initial kernel scaffold; baseline (speedup 1.0000x reference)
import jax
import jax.numpy as jnp
from jax.experimental import pallas as pl


def kernel(x, segment_label, seg_table):
    raise NotImplementedError("write your pallas kernel here")


# speedup vs baseline: 4.3065x; 4.3065x over previous
import functools

import numpy as np
import jax
import jax.numpy as jnp
from jax import lax
from jax.experimental import pallas as pl
from jax.experimental.pallas import tpu as pltpu
from jax.experimental.pallas import tpu_sc as plsc

_NC = 2
_NS = 16
_NW = _NC * _NS
_LANES = 16


def _pe2_np(max_len, L, D):
    pos = np.arange(max_len)[:, None].astype(np.float32)
    div = np.exp(np.arange(0, D, 2).astype(np.float32) * (-np.log(10000.0) / D))
    pe = np.zeros((max_len, D), np.float32)
    pe[:, 0::2] = np.sin(pos * div)
    pe[:, 1::2] = np.cos(pos * div)
    pe = pe[:L]
    return pe + pe


def _build_table(pe2, seg):
    S, D = seg.shape
    L = pe2.shape[0]

    def body(pe2_ref, seg_ref, out_ref):
        out_ref[...] = seg_ref[...][:, None, :] + pe2_ref[...][None, :, :]

    return pl.pallas_call(
        body,
        out_shape=jax.ShapeDtypeStruct((S, L, D), jnp.float32),
    )(pe2, seg)


@functools.partial(jax.jit, static_argnames=("L",))
def _sc_lookup(table, labels, L):
    N = labels.shape[0]
    D = table.shape[1]
    n_w = N // _NW
    CH = 128
    n_ch = n_w // CH
    GRP = CH // _LANES

    mesh = plsc.VectorSubcoreMesh(
        core_axis_name="c", subcore_axis_name="s",
        num_cores=_NC, num_subcores=_NS)

    @functools.partial(
        pl.kernel,
        out_type=jax.ShapeDtypeStruct((N, D), jnp.float32),
        mesh=mesh,
        scratch_types=[
            pltpu.VMEM((n_w,), jnp.int32),
            pltpu.VMEM((n_ch, CH), jnp.int32),
            pltpu.VMEM((CH, D), jnp.float32),
            pltpu.SemaphoreType.DMA,
        ],
    )
    def k(table_hbm, labels_hbm, out_hbm, lab_v, idx_v, rows_v, gsem):
        wid = lax.axis_index("s") * _NC + lax.axis_index("c")
        base = wid * n_w
        pltpu.sync_copy(labels_hbm.at[pl.ds(base, n_w)], lab_v)

        def chunk(c, carry):
            for g in range(GRP):
                off = c * CH + g * _LANES
                lab = lab_v[pl.ds(off, _LANES)]
                p = base + off + lax.iota(jnp.int32, _LANES)
                idx_v[c, pl.ds(g * _LANES, _LANES)] = lab * L + lax.rem(p, L)
            pltpu.async_copy(table_hbm.at[idx_v.at[c]], rows_v, gsem).wait()
            pltpu.sync_copy(rows_v, out_hbm.at[pl.ds(base + c * CH, CH)])
            return carry

        lax.fori_loop(0, n_ch, chunk, 0)

    return k(table, labels)


def kernel(x, segment_label, seg_table):
    B, L = segment_label.shape
    S, D = seg_table.shape
    pe2 = jnp.asarray(_pe2_np(512, L, D))
    table = _build_table(pe2, seg_table).reshape(S * L, D)
    labels = segment_label.reshape(B * L)
    out = _sc_lookup(table, labels, L)
    return out.reshape(B, L, D)


# speedup vs baseline: 4.6567x; 1.0813x over previous
import functools

import numpy as np
import jax
import jax.numpy as jnp
from jax import lax
from jax.experimental import pallas as pl
from jax.experimental.pallas import tpu as pltpu
from jax.experimental.pallas import tpu_sc as plsc

_NC = 2
_NS = 16
_NW = _NC * _NS
_LANES = 16


def _pe2_np(max_len, L, D):
    pos = np.arange(max_len)[:, None].astype(np.float32)
    div = np.exp(np.arange(0, D, 2).astype(np.float32) * (-np.log(10000.0) / D))
    pe = np.zeros((max_len, D), np.float32)
    pe[:, 0::2] = np.sin(pos * div)
    pe[:, 1::2] = np.cos(pos * div)
    pe = pe[:L]
    return pe + pe


def _build_table(pe2, seg):
    S, D = seg.shape
    L = pe2.shape[0]

    def body(pe2_ref, seg_ref, out_ref):
        out_ref[...] = seg_ref[...][:, None, :] + pe2_ref[...][None, :, :]

    return pl.pallas_call(
        body,
        out_shape=jax.ShapeDtypeStruct((S, L, D), jnp.float32),
    )(pe2, seg)


@functools.partial(jax.jit, static_argnames=("L",))
def _sc_lookup(table, labels, L):
    N = labels.shape[0]
    D = table.shape[1]
    n_w = N // _NW
    CH = 128
    n_ch = n_w // CH
    GRP = CH // _LANES

    mesh = plsc.VectorSubcoreMesh(
        core_axis_name="c", subcore_axis_name="s",
        num_cores=_NC, num_subcores=_NS)

    @functools.partial(
        pl.kernel,
        out_type=jax.ShapeDtypeStruct((N, D), jnp.float32),
        mesh=mesh,
        scratch_types=[
            pltpu.VMEM((n_w,), jnp.int32),
            pltpu.VMEM((n_ch, CH), jnp.int32),
            pltpu.VMEM((CH, D), jnp.float32),
            pltpu.VMEM((CH, D), jnp.float32),
            pltpu.SemaphoreType.DMA,
            pltpu.SemaphoreType.DMA,
            pltpu.SemaphoreType.DMA,
            pltpu.SemaphoreType.DMA,
        ],
    )
    def k(table_hbm, labels_hbm, out_hbm, lab_v, idx_v,
          rows0, rows1, gs0, gs1, ws0, ws1):
        rows, gs, ws = (rows0, rows1), (gs0, gs1), (ws0, ws1)
        wid = lax.axis_index("s") * _NC + lax.axis_index("c")
        base = wid * n_w
        pltpu.sync_copy(labels_hbm.at[pl.ds(base, n_w)], lab_v)
        iota = lax.iota(jnp.int32, _LANES)

        def compute_idx(c):
            for g in range(GRP):
                off = c * CH + g * _LANES
                lab = lab_v[pl.ds(off, _LANES)]
                p = base + off + iota
                idx_v[c, pl.ds(g * _LANES, _LANES)] = lab * L + lax.rem(p, L)

        def g_copy(c, b):
            return pltpu.make_async_copy(
                table_hbm.at[idx_v.at[c]], rows[b], gs[b])

        def w_copy(c, b):
            return pltpu.make_async_copy(
                rows[b], out_hbm.at[pl.ds(base + c * CH, CH)], ws[b])

        for b in range(2):
            compute_idx(b)
            g_copy(b, b).start()

        def body(i, carry):
            c = 2 * i
            for b in range(2):
                cc = c + b
                g_copy(cc, b).wait()
                w_copy(cc, b).start()
                compute_idx(cc + 2)
                w_copy(cc, b).wait()
                g_copy(cc + 2, b).start()
            return carry

        lax.fori_loop(0, (n_ch - 2) // 2, body, 0)

        for b in range(2):
            cc = n_ch - 2 + b
            g_copy(cc, b).wait()
            w_copy(cc, b).start()
            w_copy(cc, b).wait()

    return k(table, labels)


def kernel(x, segment_label, seg_table):
    B, L = segment_label.shape
    S, D = seg_table.shape
    pe2 = jnp.asarray(_pe2_np(512, L, D))
    table = _build_table(pe2, seg_table).reshape(S * L, D)
    labels = segment_label.reshape(B * L)
    out = _sc_lookup(table, labels, L)
    return out.reshape(B, L, D)


# speedup vs baseline: 11.5337x; 2.4768x over previous
import functools

import numpy as np
import jax
import jax.numpy as jnp
from jax import lax
from jax.experimental import pallas as pl
from jax.experimental.pallas import tpu as pltpu
from jax.experimental.pallas import tpu_sc as plsc

_NC = 2
_NS = 16
_NW = _NC * _NS
_LANES = 16


def _pe2_np(max_len, L, D):
    pos = np.arange(max_len)[:, None].astype(np.float32)
    div = np.exp(np.arange(0, D, 2).astype(np.float32) * (-np.log(10000.0) / D))
    pe = np.zeros((max_len, D), np.float32)
    pe[:, 0::2] = np.sin(pos * div)
    pe[:, 1::2] = np.cos(pos * div)
    pe = pe[:L]
    return pe + pe


def _build_table(pe2, seg):
    S, D = seg.shape
    L = pe2.shape[0]

    def body(pe2_ref, seg_ref, out_ref):
        out_ref[...] = seg_ref[...][:, None, :] + pe2_ref[...][None, :, :]

    return pl.pallas_call(
        body,
        out_shape=jax.ShapeDtypeStruct((S, L, D), jnp.float32),
    )(pe2, seg)


@functools.partial(jax.jit, static_argnames=("L",))
def _sc_lookup(table, labels, L):
    N = labels.shape[0]
    D = table.shape[1]
    n_w = N // _NW
    CH = 128
    n_ch = n_w // CH
    GRP = CH // _LANES

    mesh = plsc.VectorSubcoreMesh(
        core_axis_name="c", subcore_axis_name="s",
        num_cores=_NC, num_subcores=_NS)

    @functools.partial(
        pl.kernel,
        out_type=jax.ShapeDtypeStruct((N, D), jnp.float32),
        mesh=mesh,
        scratch_types=[
            pltpu.VMEM_SHARED(table.shape, jnp.float32),
            pltpu.VMEM((n_w,), jnp.int32),
            pltpu.VMEM((n_ch, CH), jnp.int32),
            pltpu.VMEM((CH, D), jnp.float32),
            pltpu.VMEM((CH, D), jnp.float32),
            pltpu.SemaphoreType.DMA,
            pltpu.SemaphoreType.DMA,
            pltpu.SemaphoreType.DMA,
            pltpu.SemaphoreType.DMA,
        ],
    )
    def k(table_hbm, labels_hbm, out_hbm, table_sh, lab_v, idx_v,
          rows0, rows1, gs0, gs1, ws0, ws1):
        rows, gs, ws = (rows0, rows1), (gs0, gs1), (ws0, ws1)
        sid = lax.axis_index("s")
        wid = sid * _NC + lax.axis_index("c")
        base = wid * n_w
        @pl.when(sid == 0)
        def _():
            pltpu.sync_copy(table_hbm, table_sh)
        pltpu.sync_copy(labels_hbm.at[pl.ds(base, n_w)], lab_v)
        plsc.subcore_barrier()
        iota = lax.iota(jnp.int32, _LANES)

        def compute_idx(c):
            for g in range(GRP):
                off = c * CH + g * _LANES
                lab = lab_v[pl.ds(off, _LANES)]
                p = base + off + iota
                idx_v[c, pl.ds(g * _LANES, _LANES)] = lab * L + lax.rem(p, L)

        def g_copy(c, b):
            return pltpu.make_async_copy(
                table_sh.at[idx_v.at[c]], rows[b], gs[b])

        def w_copy(c, b):
            return pltpu.make_async_copy(
                rows[b], out_hbm.at[pl.ds(base + c * CH, CH)], ws[b])

        for b in range(2):
            compute_idx(b)
            g_copy(b, b).start()

        def body(i, carry):
            c = 2 * i
            for b in range(2):
                cc = c + b
                g_copy(cc, b).wait()
                w_copy(cc, b).start()
                compute_idx(cc + 2)
                w_copy(cc, b).wait()
                g_copy(cc + 2, b).start()
            return carry

        lax.fori_loop(0, (n_ch - 2) // 2, body, 0)

        for b in range(2):
            cc = n_ch - 2 + b
            g_copy(cc, b).wait()
            w_copy(cc, b).start()
            w_copy(cc, b).wait()

    return k(table, labels)


def kernel(x, segment_label, seg_table):
    B, L = segment_label.shape
    S, D = seg_table.shape
    pe2 = jnp.asarray(_pe2_np(512, L, D))
    table = _build_table(pe2, seg_table).reshape(S * L, D)
    labels = segment_label.reshape(B * L)
    out = _sc_lookup(table, labels, L)
    return out.reshape(B, L, D)
